# Initial kernel scaffold; baseline (speedup 1.0000x reference)
import jax
import jax.numpy as jnp
from jax.experimental import pallas as pl


def kernel(x, edge_index, W1, b1, W2, b2, W3, b3, W4, b4):
    raise NotImplementedError("write your pallas kernel here")


# speedup vs baseline: 22.4154x; 22.4154x over previous
import functools

import jax
import jax.numpy as jnp
from jax import lax
from jax.experimental import pallas as pl
from jax.experimental.pallas import tpu as pltpu, tpu_sc as plsc

N = 10000
E = 320000
D = 128
H = 16
C = 5
CP = 8

NPAD = 10240
NC = 2
NS = 16
NW = NC * NS
EW = E // NW
CHUNK = 128
NFULL = EW // CHUNK
TAIL = EW - NFULL * CHUNK
RPT = NPAD // NS


def _fill(ref, n, val):
    v = jnp.full((16,), val, jnp.float32)

    def body(i, c):
        ref[pl.ds(i * 16, 16)] = v
        return c

    lax.fori_loop(0, n // 16, body, 0)


def _make_deg_kernel():
    mesh = plsc.VectorSubcoreMesh(core_axis_name="c", subcore_axis_name="s")

    @functools.partial(
        pl.kernel,
        mesh=mesh,
        out_type=jax.ShapeDtypeStruct((NC, NPAD), jnp.float32),
        scratch_types=[
            pltpu.VMEM((CHUNK,), jnp.int32),
            pltpu.VMEM((TAIL,), jnp.int32),
            pltpu.VMEM((CHUNK,), jnp.float32),
            pltpu.VMEM((RPT,), jnp.float32),
            pltpu.VMEM_SHARED((NPAD,), jnp.float32),
        ],
    )
    def deg_kernel(ei, out, dst_v, dst_t, ones_v, z_v, acc):
        cid = lax.axis_index("c")
        sid = lax.axis_index("s")
        wid = cid * NS + sid
        base = wid * EW

        _fill(z_v, RPT, 0.0)
        _fill(ones_v, CHUNK, 1.0)
        pltpu.sync_copy(z_v, acc.at[pl.ds(sid * RPT, RPT)])
        plsc.subcore_barrier()

        def body(j, c):
            pltpu.sync_copy(ei.at[pl.ds(E + base + j * CHUNK, CHUNK)], dst_v)
            pltpu.sync_copy(ones_v, acc.at[dst_v], add=True)
            return c

        lax.fori_loop(0, NFULL, body, 0)
        pltpu.sync_copy(ei.at[pl.ds(E + base + NFULL * CHUNK, TAIL)], dst_t)
        pltpu.sync_copy(ones_v.at[pl.ds(0, TAIL)], acc.at[dst_t], add=True)

        plsc.subcore_barrier()
        pltpu.sync_copy(
            acc.at[pl.ds(sid * RPT, RPT)],
            out.at[cid, pl.ds(sid * RPT, RPT)],
        )

    return deg_kernel


def _make_edge_kernel(width):
    mesh = plsc.VectorSubcoreMesh(core_axis_name="c", subcore_axis_name="s")

    @functools.partial(
        pl.kernel,
        mesh=mesh,
        out_type=jax.ShapeDtypeStruct((NC, NPAD, width), jnp.float32),
        scratch_types=[
            pltpu.VMEM((CHUNK,), jnp.int32),
            pltpu.VMEM((CHUNK,), jnp.int32),
            pltpu.VMEM((TAIL,), jnp.int32),
            pltpu.VMEM((TAIL,), jnp.int32),
            pltpu.VMEM((CHUNK, width), jnp.float32),
            pltpu.VMEM((TAIL, width), jnp.float32),
            pltpu.VMEM_SHARED((NPAD, width), jnp.float32),
            pltpu.SemaphoreType.DMA,
        ],
        compiler_params=pltpu.CompilerParams(use_tc_tiling_on_sc=False),
    )
    def edge_kernel(ei, g, zeros, out, src_v, dst_v, src_t, dst_t, rows_v,
                    rows_t, acc, sem):
        cid = lax.axis_index("c")
        sid = lax.axis_index("s")
        wid = cid * NS + sid
        base = wid * EW

        pltpu.sync_copy(
            zeros.at[pl.ds(sid * RPT, RPT)],
            acc.at[pl.ds(sid * RPT, RPT)],
        )
        plsc.subcore_barrier()

        def body(j, c):
            pltpu.sync_copy(ei.at[pl.ds(base + j * CHUNK, CHUNK)], src_v)
            pltpu.sync_copy(ei.at[pl.ds(E + base + j * CHUNK, CHUNK)], dst_v)
            pltpu.async_copy(g.at[src_v], rows_v, sem).wait()
            pltpu.sync_copy(rows_v, acc.at[dst_v], add=True)
            return c

        lax.fori_loop(0, NFULL, body, 0)
        tb = base + NFULL * CHUNK
        pltpu.sync_copy(ei.at[pl.ds(tb, TAIL)], src_t)
        pltpu.sync_copy(ei.at[pl.ds(E + tb, TAIL)], dst_t)
        pltpu.async_copy(g.at[src_t], rows_t, sem).wait()
        pltpu.sync_copy(rows_t, acc.at[dst_t], add=True)

        plsc.subcore_barrier()
        pltpu.sync_copy(
            acc.at[pl.ds(sid * RPT, RPT)],
            out.at[cid, pl.ds(sid * RPT, RPT)],
        )

    return edge_kernel


def _k2_body(x_ref, w1_ref, degp_ref, g1_ref, dinv_ref):
    deg = degp_ref[0:N, :] + degp_ref[NPAD:NPAD + N, :] + 1.0
    dinv = lax.rsqrt(deg)
    h1 = jnp.dot(x_ref[...], w1_ref[...], preferred_element_type=jnp.float32)
    g1_ref[...] = h1 * dinv
    dinv_ref[...] = dinv


def _k4_body(p_ref, g1_ref, dinv_ref, b1_ref, w2_ref, g2_ref):
    dinv = dinv_ref[...]
    s = p_ref[0:N, :] + p_ref[NPAD:NPAD + N, :] + g1_ref[...]
    r1 = jnp.maximum(dinv * s + b1_ref[...], 0.0)
    h2 = jnp.dot(r1, w2_ref[...], preferred_element_type=jnp.float32)
    g2_ref[...] = h2 * dinv


def _k6_body(q_ref, g2_ref, dinv_ref, b2_ref, w3_ref, b3_ref, w4_ref, b4_ref,
             out_ref):
    dinv = dinv_ref[...]
    s = q_ref[0:N, :] + q_ref[NPAD:NPAD + N, :] + g2_ref[...]
    agg2 = dinv * s + b2_ref[...]
    z1 = jnp.maximum(
        jnp.dot(agg2, w3_ref[...], preferred_element_type=jnp.float32)
        + b3_ref[...], 0.0)
    z = jnp.dot(z1, w4_ref[...], preferred_element_type=jnp.float32) \
        + b4_ref[...]
    m = jnp.max(z, axis=1, keepdims=True)
    lse = jnp.log(jnp.sum(jnp.exp(z - m), axis=1, keepdims=True)) + m
    out_ref[...] = z - lse


def kernel(x, edge_index, W1, b1, W2, b2, W3, b3, W4, b4):
    deg_k = _make_deg_kernel()
    edge16 = _make_edge_kernel(H)
    edge8 = _make_edge_kernel(CP)

    ei_flat = edge_index.reshape(2 * E)

    degp = deg_k(ei_flat).reshape(NC * NPAD, 1)

    g1, dinv = pl.pallas_call(
        _k2_body,
        out_shape=[
            jax.ShapeDtypeStruct((N, H), jnp.float32),
            jax.ShapeDtypeStruct((N, 1), jnp.float32),
        ],
    )(x, W1, degp)

    z16 = jnp.zeros((NPAD, H), jnp.float32)
    p1 = edge16(ei_flat, g1, z16).reshape(NC * NPAD, H)

    W2p = jnp.concatenate([W2, jnp.zeros((H, CP - C), jnp.float32)], axis=1)
    g2 = pl.pallas_call(
        _k4_body,
        out_shape=jax.ShapeDtypeStruct((N, CP), jnp.float32),
    )(p1, g1, dinv, b1.reshape(1, H), W2p)

    z8 = jnp.zeros((NPAD, CP), jnp.float32)
    p2 = edge8(ei_flat, g2, z8).reshape(NC * NPAD, CP)

    b2p = jnp.concatenate([b2, jnp.zeros((CP - C,), jnp.float32)])
    W3p = jnp.concatenate([W3, jnp.zeros((CP - C, 32), jnp.float32)], axis=0)
    out = pl.pallas_call(
        _k6_body,
        out_shape=jax.ShapeDtypeStruct((N, C), jnp.float32),
    )(p2, g2, dinv, b2p.reshape(1, CP), W3p, b3.reshape(1, 32), W4,
      b4.reshape(1, C))
    return out


# speedup vs baseline: 59.3879x; 2.6494x over previous
import functools

import jax
import jax.numpy as jnp
from jax import lax
from jax.experimental import pallas as pl
from jax.experimental.pallas import tpu as pltpu, tpu_sc as plsc

N = 10000
E = 320000
D = 128
H = 16
C = 5
CP = 8

NPAD = 10240
NC = 2
NS = 16
NW = NC * NS
EW = E // NW
CHUNK = 2000
NFULL = EW // CHUNK
RPT = NPAD // NS


def _fill(ref, n, val):
    v = jnp.full((16,), val, jnp.float32)

    def body(i, c):
        ref[pl.ds(i * 16, 16)] = v
        return c

    lax.fori_loop(0, n // 16, body, 0)


def _make_deg_kernel():
    mesh = plsc.VectorSubcoreMesh(core_axis_name="c", subcore_axis_name="s")

    @functools.partial(
        pl.kernel,
        mesh=mesh,
        out_type=jax.ShapeDtypeStruct((NC, NPAD), jnp.float32),
        scratch_types=[
            pltpu.VMEM((CHUNK,), jnp.int32),
            pltpu.VMEM((CHUNK,), jnp.float32),
            pltpu.VMEM((RPT,), jnp.float32),
            pltpu.VMEM_SHARED((NPAD,), jnp.float32),
        ],
    )
    def deg_kernel(ei, out, dst_v, ones_v, z_v, acc):
        cid = lax.axis_index("c")
        sid = lax.axis_index("s")
        wid = cid * NS + sid
        base = wid * EW

        _fill(z_v, RPT, 0.0)
        _fill(ones_v, CHUNK, 1.0)
        pltpu.sync_copy(z_v, acc.at[pl.ds(sid * RPT, RPT)])
        plsc.subcore_barrier()

        def body(j, c):
            pltpu.sync_copy(ei.at[pl.ds(E + base + j * CHUNK, CHUNK)], dst_v)
            pltpu.sync_copy(ones_v, acc.at[dst_v], add=True)
            return c

        lax.fori_loop(0, NFULL, body, 0)

        plsc.subcore_barrier()
        pltpu.sync_copy(
            acc.at[pl.ds(sid * RPT, RPT)],
            out.at[cid, pl.ds(sid * RPT, RPT)],
        )

    return deg_kernel


def _make_edge_kernel(width):
    mesh = plsc.VectorSubcoreMesh(core_axis_name="c", subcore_axis_name="s")

    @functools.partial(
        pl.kernel,
        mesh=mesh,
        out_type=jax.ShapeDtypeStruct((NC, NPAD, width), jnp.float32),
        scratch_types=[
            pltpu.VMEM((CHUNK,), jnp.int32),
            pltpu.VMEM((CHUNK,), jnp.int32),
            pltpu.VMEM((CHUNK, width), jnp.float32),
            pltpu.VMEM_SHARED((NPAD, width), jnp.float32),
            pltpu.SemaphoreType.DMA,
        ],
        compiler_params=pltpu.CompilerParams(use_tc_tiling_on_sc=False),
    )
    def edge_kernel(ei, g, zeros, out, src_v, dst_v, rows_v, acc, sem):
        cid = lax.axis_index("c")
        sid = lax.axis_index("s")
        wid = cid * NS + sid
        base = wid * EW

        pltpu.sync_copy(
            zeros.at[pl.ds(sid * RPT, RPT)],
            acc.at[pl.ds(sid * RPT, RPT)],
        )
        plsc.subcore_barrier()

        def body(j, c):
            pltpu.sync_copy(ei.at[pl.ds(base + j * CHUNK, CHUNK)], src_v)
            pltpu.sync_copy(ei.at[pl.ds(E + base + j * CHUNK, CHUNK)], dst_v)
            pltpu.async_copy(g.at[src_v], rows_v, sem).wait()
            pltpu.sync_copy(rows_v, acc.at[dst_v], add=True)
            return c

        lax.fori_loop(0, NFULL, body, 0)

        plsc.subcore_barrier()
        pltpu.sync_copy(
            acc.at[pl.ds(sid * RPT, RPT)],
            out.at[cid, pl.ds(sid * RPT, RPT)],
        )

    return edge_kernel


def _k2_body(x_ref, w1_ref, degp_ref, g1_ref, dinv_ref):
    deg = degp_ref[0:N, :] + degp_ref[NPAD:NPAD + N, :] + 1.0
    dinv = lax.rsqrt(deg)
    h1 = jnp.dot(x_ref[...], w1_ref[...], preferred_element_type=jnp.float32)
    g1_ref[...] = h1 * dinv
    dinv_ref[...] = dinv


def _k4_body(p_ref, g1_ref, dinv_ref, b1_ref, w2_ref, g2_ref):
    dinv = dinv_ref[...]
    s = p_ref[0:N, :] + p_ref[NPAD:NPAD + N, :] + g1_ref[...]
    r1 = jnp.maximum(dinv * s + b1_ref[...], 0.0)
    h2 = jnp.dot(r1, w2_ref[...], preferred_element_type=jnp.float32)
    g2_ref[...] = h2 * dinv


def _k6_body(q_ref, g2_ref, dinv_ref, b2_ref, w3_ref, b3_ref, w4_ref, b4_ref,
             out_ref):
    dinv = dinv_ref[...]
    s = q_ref[0:N, :] + q_ref[NPAD:NPAD + N, :] + g2_ref[...]
    agg2 = dinv * s + b2_ref[...]
    z1 = jnp.maximum(
        jnp.dot(agg2, w3_ref[...], preferred_element_type=jnp.float32)
        + b3_ref[...], 0.0)
    z = jnp.dot(z1, w4_ref[...], preferred_element_type=jnp.float32) \
        + b4_ref[...]
    m = jnp.max(z, axis=1, keepdims=True)
    lse = jnp.log(jnp.sum(jnp.exp(z - m), axis=1, keepdims=True)) + m
    out_ref[...] = z - lse


def kernel(x, edge_index, W1, b1, W2, b2, W3, b3, W4, b4):
    deg_k = _make_deg_kernel()
    edge16 = _make_edge_kernel(H)
    edge8 = _make_edge_kernel(CP)

    ei_flat = edge_index.reshape(2 * E)

    degp = deg_k(ei_flat).reshape(NC * NPAD, 1)

    g1, dinv = pl.pallas_call(
        _k2_body,
        out_shape=[
            jax.ShapeDtypeStruct((N, H), jnp.float32),
            jax.ShapeDtypeStruct((N, 1), jnp.float32),
        ],
    )(x, W1, degp)

    z16 = jnp.zeros((NPAD, H), jnp.float32)
    p1 = edge16(ei_flat, g1, z16).reshape(NC * NPAD, H)

    W2p = jnp.concatenate([W2, jnp.zeros((H, CP - C), jnp.float32)], axis=1)
    g2 = pl.pallas_call(
        _k4_body,
        out_shape=jax.ShapeDtypeStruct((N, CP), jnp.float32),
    )(p1, g1, dinv, b1.reshape(1, H), W2p)

    z8 = jnp.zeros((NPAD, CP), jnp.float32)
    p2 = edge8(ei_flat, g2, z8).reshape(NC * NPAD, CP)

    b2p = jnp.concatenate([b2, jnp.zeros((CP - C,), jnp.float32)])
    W3p = jnp.concatenate([W3, jnp.zeros((CP - C, 32), jnp.float32)], axis=0)
    out = pl.pallas_call(
        _k6_body,
        out_shape=jax.ShapeDtypeStruct((N, C), jnp.float32),
    )(p2, g2, dinv, b2p.reshape(1, CP), W3p, b3.reshape(1, 32), W4,
      b4.reshape(1, C))
    return out


# speedup vs baseline: 63.2935x; 1.0658x over previous
import functools

import jax
import jax.numpy as jnp
from jax import lax
from jax.experimental import pallas as pl
from jax.experimental.pallas import tpu as pltpu, tpu_sc as plsc

N = 10000
E = 320000
D = 128
H = 16
C = 5
CP = 8

NPAD = 10240
NC = 2
NS = 16
NW = NC * NS
EW = E // NW
DCHUNK = EW
CHUNK = 5000
NFULL = EW // CHUNK
RPT = NPAD // NS


def _fill(ref, n, val):
    v = jnp.full((16,), val, jnp.float32)

    def body(i, c):
        ref[pl.ds(i * 16, 16)] = v
        return c

    lax.fori_loop(0, n // 16, body, 0)


def _make_deg_kernel():
    mesh = plsc.VectorSubcoreMesh(core_axis_name="c", subcore_axis_name="s")

    @functools.partial(
        pl.kernel,
        mesh=mesh,
        out_type=jax.ShapeDtypeStruct((NC, NPAD), jnp.float32),
        scratch_types=[
            pltpu.VMEM((DCHUNK,), jnp.int32),
            pltpu.VMEM((DCHUNK,), jnp.float32),
            pltpu.VMEM((RPT,), jnp.float32),
            pltpu.VMEM_SHARED((NPAD,), jnp.float32),
        ],
    )
    def deg_kernel(ei, out, dst_v, ones_v, z_v, acc):
        cid = lax.axis_index("c")
        sid = lax.axis_index("s")
        wid = cid * NS + sid
        base = wid * EW

        _fill(z_v, RPT, 0.0)
        _fill(ones_v, DCHUNK, 1.0)
        pltpu.sync_copy(z_v, acc.at[pl.ds(sid * RPT, RPT)])
        plsc.subcore_barrier()

        pltpu.sync_copy(ei.at[pl.ds(E + base, DCHUNK)], dst_v)
        pltpu.sync_copy(ones_v, acc.at[dst_v], add=True)

        plsc.subcore_barrier()
        pltpu.sync_copy(
            acc.at[pl.ds(sid * RPT, RPT)],
            out.at[cid, pl.ds(sid * RPT, RPT)],
        )

    return deg_kernel


def _make_edge_kernel(width):
    mesh = plsc.VectorSubcoreMesh(core_axis_name="c", subcore_axis_name="s")

    @functools.partial(
        pl.kernel,
        mesh=mesh,
        out_type=jax.ShapeDtypeStruct((NC, NPAD, width), jnp.float32),
        scratch_types=[
            pltpu.VMEM((CHUNK,), jnp.int32),
            pltpu.VMEM((CHUNK,), jnp.int32),
            pltpu.VMEM((CHUNK, width), jnp.float32),
            pltpu.VMEM_SHARED((NPAD, width), jnp.float32),
            pltpu.SemaphoreType.DMA,
        ],
        compiler_params=pltpu.CompilerParams(use_tc_tiling_on_sc=False),
    )
    def edge_kernel(ei, g, zeros, out, src_v, dst_v, rows_v, acc, sem):
        cid = lax.axis_index("c")
        sid = lax.axis_index("s")
        wid = cid * NS + sid
        base = wid * EW

        pltpu.sync_copy(
            zeros.at[pl.ds(sid * RPT, RPT)],
            acc.at[pl.ds(sid * RPT, RPT)],
        )
        plsc.subcore_barrier()

        def body(j, c):
            pltpu.sync_copy(ei.at[pl.ds(base + j * CHUNK, CHUNK)], src_v)
            pltpu.sync_copy(ei.at[pl.ds(E + base + j * CHUNK, CHUNK)], dst_v)
            pltpu.async_copy(g.at[src_v], rows_v, sem).wait()
            pltpu.sync_copy(rows_v, acc.at[dst_v], add=True)
            return c

        lax.fori_loop(0, NFULL, body, 0)

        plsc.subcore_barrier()
        pltpu.sync_copy(
            acc.at[pl.ds(sid * RPT, RPT)],
            out.at[cid, pl.ds(sid * RPT, RPT)],
        )

    return edge_kernel


def _k2_body(x_ref, w1_ref, degp_ref, g1_ref, dinv_ref):
    deg = degp_ref[0:N, :] + degp_ref[NPAD:NPAD + N, :] + 1.0
    dinv = lax.rsqrt(deg)
    h1 = jnp.dot(x_ref[...], w1_ref[...], preferred_element_type=jnp.float32)
    g1_ref[...] = h1 * dinv
    dinv_ref[...] = dinv


def _k4_body(p_ref, g1_ref, dinv_ref, b1_ref, w2_ref, g2_ref):
    dinv = dinv_ref[...]
    s = p_ref[0:N, :] + p_ref[NPAD:NPAD + N, :] + g1_ref[...]
    r1 = jnp.maximum(dinv * s + b1_ref[...], 0.0)
    h2 = jnp.dot(r1, w2_ref[...], preferred_element_type=jnp.float32)
    g2_ref[...] = h2 * dinv


def _k6_body(q_ref, g2_ref, dinv_ref, b2_ref, w3_ref, b3_ref, w4_ref, b4_ref,
             out_ref):
    dinv = dinv_ref[...]
    s = q_ref[0:N, :] + q_ref[NPAD:NPAD + N, :] + g2_ref[...]
    agg2 = dinv * s + b2_ref[...]
    z1 = jnp.maximum(
        jnp.dot(agg2, w3_ref[...], preferred_element_type=jnp.float32)
        + b3_ref[...], 0.0)
    z = jnp.dot(z1, w4_ref[...], preferred_element_type=jnp.float32) \
        + b4_ref[...]
    m = jnp.max(z, axis=1, keepdims=True)
    lse = jnp.log(jnp.sum(jnp.exp(z - m), axis=1, keepdims=True)) + m
    out_ref[...] = z - lse


def kernel(x, edge_index, W1, b1, W2, b2, W3, b3, W4, b4):
    deg_k = _make_deg_kernel()
    edge16 = _make_edge_kernel(H)
    edge8 = _make_edge_kernel(CP)

    ei_flat = edge_index.reshape(2 * E)

    degp = deg_k(ei_flat).reshape(NC * NPAD, 1)

    g1, dinv = pl.pallas_call(
        _k2_body,
        out_shape=[
            jax.ShapeDtypeStruct((N, H), jnp.float32),
            jax.ShapeDtypeStruct((N, 1), jnp.float32),
        ],
    )(x, W1, degp)

    z16 = jnp.zeros((NPAD, H), jnp.float32)
    p1 = edge16(ei_flat, g1, z16).reshape(NC * NPAD, H)

    W2p = jnp.concatenate([W2, jnp.zeros((H, CP - C), jnp.float32)], axis=1)
    g2 = pl.pallas_call(
        _k4_body,
        out_shape=jax.ShapeDtypeStruct((N, CP), jnp.float32),
    )(p1, g1, dinv, b1.reshape(1, H), W2p)

    z8 = jnp.zeros((NPAD, CP), jnp.float32)
    p2 = edge8(ei_flat, g2, z8).reshape(NC * NPAD, CP)

    b2p = jnp.concatenate([b2, jnp.zeros((CP - C,), jnp.float32)])
    W3p = jnp.concatenate([W3, jnp.zeros((CP - C, 32), jnp.float32)], axis=0)
    out = pl.pallas_call(
        _k6_body,
        out_shape=jax.ShapeDtypeStruct((N, C), jnp.float32),
    )(p2, g2, dinv, b2p.reshape(1, CP), W3p, b3.reshape(1, 32), W4,
      b4.reshape(1, C))
    return out
